# baseline (speedup 1.0000x reference)
import functools

import jax
import jax.numpy as jnp
from jax import lax
from jax.experimental import pallas as pl
from jax.experimental.pallas import tpu as pltpu
from jax.experimental.pallas import tpu_sc as plsc

N = 10000
E = 320000
D_IN = 128
D = 16
N_CLASSES = 40

NC = 2
NS = 16
NW = NC * NS
C = 128
SK = 8
SUPER = SK * C
ROUNDS = 10
NCHUNK = E // C
FULL_SUP = E // SUPER
TAIL_SK = NCHUNK - FULL_SUP * SK
TAIL_VALID = TAIL_SK * C
N_PAD = 10240
ROWS_PER_TILE = N_PAD // NS
EA_R = E * D // 128
SLAB = 16
NSLAB = EA_R // SLAB
CROUNDS = (NSLAB + NW - 1) // NW

_mesh = plsc.VectorSubcoreMesh(
    core_axis_name="c", subcore_axis_name="s", num_cores=NC, num_subcores=NS
)


@functools.partial(
    pl.kernel,
    out_type=jax.ShapeDtypeStruct((NC, N_PAD, D), jnp.float32),
    mesh=_mesh,
    scratch_types=[
        pltpu.VMEM((3, SK, C), jnp.int32),
        pltpu.VMEM((4, SK, C), jnp.int32),
        pltpu.VMEM((3, SUPER * D // 128, 128), jnp.float32),
        pltpu.VMEM((3, SUPER, D), jnp.float32),
        pltpu.VMEM_SHARED((N_PAD, D), jnp.float32),
        pltpu.SemaphoreType.DMA,
        pltpu.SemaphoreType.DMA,
        pltpu.SemaphoreType.DMA,
        pltpu.SemaphoreType.DMA,
        pltpu.SemaphoreType.DMA,
        pltpu.SemaphoreType.DMA,
        pltpu.SemaphoreType.DMA,
    ],
    compiler_params=pltpu.CompilerParams(use_tc_tiling_on_sc=False),
)
def _sc_aggregate(h_hbm, ei_hbm, ea_hbm, out_hbm,
                  src_v, dst_v, ea_v, rows_v, acc_sh,
                  sem_i0, sem_i1, sem_i2, sem_g0, sem_g1, sem_s0, sem_s1):
    sem_i = (sem_i0, sem_i1, sem_i2)
    sem_g = (sem_g0, sem_g1)
    sem_s = (sem_s0, sem_s1)
    cid = lax.axis_index("c")
    sid = lax.axis_index("s")
    wid = sid * NC + cid

    def _zero(i, carry):
        rows_v[0, i, :] = jnp.zeros((D,), jnp.float32)
        return carry
    plsc.parallel_loop(0, ROWS_PER_TILE, 1, unroll=8,
                       carry=jnp.int32(0))(_zero)
    pltpu.sync_copy(rows_v.at[0, pl.ds(0, ROWS_PER_TILE)],
                    acc_sh.at[pl.ds(sid * ROWS_PER_TILE, ROWS_PER_TILE)])
    plsc.subcore_barrier()

    def sup_idx(j):
        return j * NW + wid

    def variants(s, emit):
        @pl.when(s < FULL_SUP)
        def _():
            emit(SK)

        @pl.when(s == FULL_SUP)
        def _():
            emit(TAIL_SK)

    def start_inputs(j, b):
        s = sup_idx(j)

        def emit(nk):
            for k in range(nk):
                off = (s * SK + k) * C
                pltpu.async_copy(ei_hbm.at[0, pl.ds(off, C)],
                                 src_v.at[b, k], sem_i[b])
                pltpu.async_copy(ei_hbm.at[1, pl.ds(off, C)],
                                 dst_v.at[j % 4, k], sem_i[b])
            pltpu.async_copy(ea_hbm.at[pl.ds(s * (SUPER * D // 128), nk * C * D // 128)],
                             ea_v.at[b, pl.ds(0, nk * C * D // 128)],
                             sem_i[b])
        variants(s, emit)

    def wait_inputs(j, b):
        s = sup_idx(j)

        def emit(nk):
            for k in range(nk):
                pltpu.make_async_copy(ei_hbm.at[0, pl.ds(0, C)],
                                      src_v.at[b, k], sem_i[b]).wait()
                pltpu.make_async_copy(ei_hbm.at[1, pl.ds(0, C)],
                                      dst_v.at[j % 4, k], sem_i[b]).wait()
            pltpu.make_async_copy(ea_hbm.at[pl.ds(0, nk * C * D // 128)],
                                  ea_v.at[b, pl.ds(0, nk * C * D // 128)],
                                  sem_i[b]).wait()
        variants(s, emit)

    def fire_gathers(j, b):
        s = sup_idx(j)

        def emit(nk):
            for k in range(nk):
                pltpu.async_copy(h_hbm.at[src_v.at[b, k]],
                                 rows_v.at[j % 3, pl.ds(k * C, C)],
                                 sem_g[j % 2])
        variants(s, emit)

    def drain_gathers(j, b):
        s = sup_idx(j)

        def emit(nk):
            for k in range(nk):
                pltpu.make_async_copy(h_hbm.at[src_v.at[b, k]],
                                      rows_v.at[j % 3, pl.ds(k * C, C)],
                                      sem_g[j % 2]).wait()
        variants(s, emit)

    def fire_scatters(j, b):
        s = sup_idx(j)

        def emit(nk):
            for k in range(nk):
                pltpu.async_copy(rows_v.at[j % 3, pl.ds(k * C, C)],
                                 acc_sh.at[dst_v.at[j % 4, k]],
                                 sem_s[j % 2], add=True)
        variants(s, emit)

    def drain_scatters(j):
        s = sup_idx(j)

        def emit(nk):
            for k in range(nk):
                pltpu.make_async_copy(rows_v.at[j % 3, pl.ds(k * C, C)],
                                      acc_sh.at[dst_v.at[j % 4, k]],
                                      sem_s[j % 2]).wait()
        variants(s, emit)

    def compute(j, b):
        def _body(g, c2):
            for rr in range(8):
                rows_v[j % 3, g * 8 + rr, :] = jnp.maximum(
                    rows_v[j % 3, g * 8 + rr, :]
                    + ea_v[b, g, pl.ds(rr * D, D)], 0.0)
            return c2
        plsc.parallel_loop(0, SUPER // 8, 1, unroll=2,
                           carry=jnp.int32(0))(_body)

    start_inputs(0, 0)
    start_inputs(1, 1)
    wait_inputs(0, 0)
    fire_gathers(0, 0)
    for j in range(ROUNDS):
        b = j % 3
        if j + 2 < ROUNDS:
            start_inputs(j + 2, (j + 2) % 3)
        drain_gathers(j, b)
        if j + 1 < ROUNDS:
            wait_inputs(j + 1, (j + 1) % 3)
            fire_gathers(j + 1, (j + 1) % 3)
        compute(j, b)
        if j - 1 >= 0:
            drain_scatters(j - 1)
        fire_scatters(j, b)
    drain_scatters(ROUNDS - 1)

    plsc.subcore_barrier()
    pltpu.sync_copy(acc_sh.at[pl.ds(sid * ROWS_PER_TILE, ROWS_PER_TILE)],
                    out_hbm.at[cid].at[pl.ds(sid * ROWS_PER_TILE, ROWS_PER_TILE)])


def _tc_embed_body(x_ref, w_ref, b_ref, o_ref):
    h = lax.dot_general(x_ref[...], w_ref[...],
                        (((1,), (1,)), ((), ())),
                        preferred_element_type=jnp.float32)
    o_ref[...] = jnp.maximum(h + b_ref[...], 0.0)


def _tc_mid_body(p_ref, h_ref, eps_ref, w_ref, b_ref, o_ref):
    agg = p_ref[0, :N, :] + p_ref[1, :N, :] + (1.0 + eps_ref[0, 0]) * h_ref[...]
    out = lax.dot_general(agg, w_ref[...], (((1,), (1,)), ((), ())),
                          preferred_element_type=jnp.float32)
    o_ref[...] = jnp.maximum(out + b_ref[...], 0.0)


def _tc_final_body(p_ref, h_ref, eps_ref, w_ref, b_ref, o_ref):
    agg = p_ref[0, :N, :] + p_ref[1, :N, :] + (1.0 + eps_ref[0, 0]) * h_ref[...]
    logits = lax.dot_general(agg, w_ref[...], (((1,), (1,)), ((), ())),
                             preferred_element_type=jnp.float32)
    logits = logits + b_ref[...]
    m = jnp.max(logits, axis=1, keepdims=True)
    z = logits - m
    lse = jnp.log(jnp.sum(jnp.exp(z), axis=1, keepdims=True))
    o_ref[...] = z - lse


def kernel(x, edge_index, edge_attr, W0, b0, eps1, W1, b1, eps2, W2, b2):
    ea_r = edge_attr.reshape(EA_R, 128)
    ea_r, x = jax.lax.optimization_barrier((ea_r, x))

    h0 = pl.pallas_call(
        _tc_embed_body,
        out_shape=jax.ShapeDtypeStruct((N, D), jnp.float32),
    )(x, W0, b0.reshape(1, D))

    p1 = _sc_aggregate(h0, edge_index, ea_r)
    h1 = pl.pallas_call(
        _tc_mid_body,
        out_shape=jax.ShapeDtypeStruct((N, D), jnp.float32),
    )(p1, h0, eps1.reshape(1, 1), W1, b1.reshape(1, D))

    p2 = _sc_aggregate(h1, edge_index, ea_r)
    out = pl.pallas_call(
        _tc_final_body,
        out_shape=jax.ShapeDtypeStruct((N, N_CLASSES), jnp.float32),
    )(p2, h1, eps2.reshape(1, 1), W2, b2.reshape(1, N_CLASSES))
    return out


# speedup vs baseline: 1.0234x; 1.0234x over previous
import functools

import jax
import jax.numpy as jnp
from jax import lax
from jax.experimental import pallas as pl
from jax.experimental.pallas import tpu as pltpu
from jax.experimental.pallas import tpu_sc as plsc

N = 10000
E = 320000
D_IN = 128
D = 16
N_CLASSES = 40

NC = 2
NS = 16
NW = NC * NS
C = 128
SK = 8
SUPER = SK * C
ROUNDS = 10
NCHUNK = E // C
FULL_SUP = E // SUPER
TAIL_SK = NCHUNK - FULL_SUP * SK
TAIL_VALID = TAIL_SK * C
N_PAD = 10240
ROWS_PER_TILE = N_PAD // NS
EA_R = E * D // 128
SLAB = 16
NSLAB = EA_R // SLAB
CROUNDS = (NSLAB + NW - 1) // NW

_mesh = plsc.VectorSubcoreMesh(
    core_axis_name="c", subcore_axis_name="s", num_cores=NC, num_subcores=NS
)


@functools.partial(
    pl.kernel,
    out_type=jax.ShapeDtypeStruct((NC, N_PAD, D), jnp.float32),
    mesh=_mesh,
    scratch_types=[
        pltpu.VMEM((3, SK, C), jnp.int32),
        pltpu.VMEM((4, SK, C), jnp.int32),
        pltpu.VMEM((3, SUPER * D // 128, 128), jnp.float32),
        pltpu.VMEM((3, SUPER, D), jnp.float32),
        pltpu.VMEM_SHARED((N_PAD, D), jnp.float32),
        pltpu.SemaphoreType.DMA,
        pltpu.SemaphoreType.DMA,
        pltpu.SemaphoreType.DMA,
        pltpu.SemaphoreType.DMA,
        pltpu.SemaphoreType.DMA,
        pltpu.SemaphoreType.DMA,
        pltpu.SemaphoreType.DMA,
    ],
    compiler_params=pltpu.CompilerParams(use_tc_tiling_on_sc=False),
)
def _sc_aggregate(h_hbm, ei_hbm, ea_hbm, out_hbm,
                  src_v, dst_v, ea_v, rows_v, acc_sh,
                  sem_i0, sem_i1, sem_i2, sem_g0, sem_g1, sem_s0, sem_s1):
    sem_i = (sem_i0, sem_i1, sem_i2)
    sem_g = (sem_g0, sem_g1)
    sem_s = (sem_s0, sem_s1)
    cid = lax.axis_index("c")
    sid = lax.axis_index("s")
    wid = sid * NC + cid

    def _zero(i, carry):
        rows_v[0, i, :] = jnp.zeros((D,), jnp.float32)
        return carry
    plsc.parallel_loop(0, ROWS_PER_TILE, 1, unroll=8,
                       carry=jnp.int32(0))(_zero)
    pltpu.sync_copy(rows_v.at[0, pl.ds(0, ROWS_PER_TILE)],
                    acc_sh.at[pl.ds(sid * ROWS_PER_TILE, ROWS_PER_TILE)])
    plsc.subcore_barrier()

    def sup_idx(j):
        return j * NW + wid

    def variants(s, emit):
        @pl.when(s < FULL_SUP)
        def _():
            emit(SK)

        @pl.when(s == FULL_SUP)
        def _():
            emit(TAIL_SK)

    def start_inputs(j, b):
        s = sup_idx(j)

        def emit(nk):
            for k in range(nk):
                off = (s * SK + k) * C
                pltpu.async_copy(ei_hbm.at[0, pl.ds(off, C)],
                                 src_v.at[b, k], sem_i[b])
                pltpu.async_copy(ei_hbm.at[1, pl.ds(off, C)],
                                 dst_v.at[j % 4, k], sem_i[b])
            pltpu.async_copy(ea_hbm.at[pl.ds(s * (SUPER * D // 128), nk * C * D // 128)],
                             ea_v.at[b, pl.ds(0, nk * C * D // 128)],
                             sem_i[b])
        variants(s, emit)

    def wait_inputs(j, b):
        s = sup_idx(j)

        def emit(nk):
            for k in range(nk):
                pltpu.make_async_copy(ei_hbm.at[0, pl.ds(0, C)],
                                      src_v.at[b, k], sem_i[b]).wait()
                pltpu.make_async_copy(ei_hbm.at[1, pl.ds(0, C)],
                                      dst_v.at[j % 4, k], sem_i[b]).wait()
            pltpu.make_async_copy(ea_hbm.at[pl.ds(0, nk * C * D // 128)],
                                  ea_v.at[b, pl.ds(0, nk * C * D // 128)],
                                  sem_i[b]).wait()
        variants(s, emit)

    def fire_gathers(j, b):
        s = sup_idx(j)

        def emit(nk):
            for k in range(nk):
                pltpu.async_copy(h_hbm.at[src_v.at[b, k]],
                                 rows_v.at[j % 3, pl.ds(k * C, C)],
                                 sem_g[j % 2])
        variants(s, emit)

    def drain_gathers(j, b):
        s = sup_idx(j)

        def emit(nk):
            for k in range(nk):
                pltpu.make_async_copy(h_hbm.at[src_v.at[b, k]],
                                      rows_v.at[j % 3, pl.ds(k * C, C)],
                                      sem_g[j % 2]).wait()
        variants(s, emit)

    def fire_scatters(j, b):
        s = sup_idx(j)

        def emit(nk):
            for k in range(nk):
                pltpu.async_copy(rows_v.at[j % 3, pl.ds(k * C, C)],
                                 acc_sh.at[dst_v.at[j % 4, k]],
                                 sem_s[j % 2], add=True)
        variants(s, emit)

    def drain_scatters(j):
        s = sup_idx(j)

        def emit(nk):
            for k in range(nk):
                pltpu.make_async_copy(rows_v.at[j % 3, pl.ds(k * C, C)],
                                      acc_sh.at[dst_v.at[j % 4, k]],
                                      sem_s[j % 2]).wait()
        variants(s, emit)

    def compute(j, b):
        def _body(g, c2):
            for rr in range(8):
                rows_v[j % 3, g * 8 + rr, :] = jnp.maximum(
                    rows_v[j % 3, g * 8 + rr, :]
                    + ea_v[b, g, pl.ds(rr * D, D)], 0.0)
            return c2
        plsc.parallel_loop(0, SUPER // 8, 1, unroll=2,
                           carry=jnp.int32(0))(_body)

    start_inputs(0, 0)
    start_inputs(1, 1)
    wait_inputs(0, 0)
    fire_gathers(0, 0)
    for j in range(ROUNDS):
        b = j % 3
        if j + 2 < ROUNDS:
            start_inputs(j + 2, (j + 2) % 3)
        drain_gathers(j, b)
        if j + 1 < ROUNDS:
            wait_inputs(j + 1, (j + 1) % 3)
            fire_gathers(j + 1, (j + 1) % 3)
        compute(j, b)
        if j - 1 >= 0:
            drain_scatters(j - 1)
        fire_scatters(j, b)
    drain_scatters(ROUNDS - 1)

    plsc.subcore_barrier()
    pltpu.sync_copy(acc_sh.at[pl.ds(sid * ROWS_PER_TILE, ROWS_PER_TILE)],
                    out_hbm.at[cid].at[pl.ds(sid * ROWS_PER_TILE, ROWS_PER_TILE)])


def _tc_embed_body(x_ref, w_ref, b_ref, o_ref):
    h = lax.dot_general(x_ref[...], w_ref[...],
                        (((1,), (1,)), ((), ())),
                        preferred_element_type=jnp.float32)
    o_ref[...] = jnp.maximum(h + b_ref[...], 0.0)


def _tc_mid_body(p_ref, h_ref, eps_ref, w_ref, b_ref, o_ref):
    agg = p_ref[0, :N, :] + p_ref[1, :N, :] + (1.0 + eps_ref[0, 0]) * h_ref[...]
    out = lax.dot_general(agg, w_ref[...], (((1,), (1,)), ((), ())),
                          preferred_element_type=jnp.float32)
    o_ref[...] = jnp.maximum(out + b_ref[...], 0.0)


def _tc_final_body(p_ref, h_ref, eps_ref, w_ref, b_ref, o_ref):
    agg = p_ref[0, :N, :] + p_ref[1, :N, :] + (1.0 + eps_ref[0, 0]) * h_ref[...]
    logits = lax.dot_general(agg, w_ref[...], (((1,), (1,)), ((), ())),
                             preferred_element_type=jnp.float32)
    logits = logits + b_ref[...]
    m = jnp.max(logits, axis=1, keepdims=True)
    z = logits - m
    lse = jnp.log(jnp.sum(jnp.exp(z), axis=1, keepdims=True))
    o_ref[...] = z - lse


def kernel(x, edge_index, edge_attr, W0, b0, eps1, W1, b1, eps2, W2, b2):
    ea_r = edge_attr.reshape(EA_R, 128)

    h0 = pl.pallas_call(
        _tc_embed_body,
        out_shape=jax.ShapeDtypeStruct((N, D), jnp.float32),
    )(x, W0, b0.reshape(1, D))

    p1 = _sc_aggregate(h0, edge_index, ea_r)
    h1 = pl.pallas_call(
        _tc_mid_body,
        out_shape=jax.ShapeDtypeStruct((N, D), jnp.float32),
    )(p1, h0, eps1.reshape(1, 1), W1, b1.reshape(1, D))

    p2 = _sc_aggregate(h1, edge_index, ea_r)
    out = pl.pallas_call(
        _tc_final_body,
        out_shape=jax.ShapeDtypeStruct((N, N_CLASSES), jnp.float32),
    )(p2, h1, eps2.reshape(1, 1), W2, b2.reshape(1, N_CLASSES))
    return out
